# baseline (speedup 1.0000x reference)
import functools

import jax
import jax.numpy as jnp
from jax.experimental import pallas as pl
from jax.experimental.pallas import tpu as pltpu

LEAKY_SLOPE = 0.1
BN_EPS = 1e-5
IMAGES_PER_STEP = 4


def _leaky(v):
    return jnp.maximum(v, LEAKY_SLOPE * v)


def _fold_kernel(*refs, n_convs, cat_pairs, f32_bias):
    ins = refs[:5 * n_convs]
    outs = refs[5 * n_convs:]
    folded = []
    for i in range(n_convs):
        w, g, b, m, v = ins[5 * i:5 * i + 5]
        scale = g[...] * jax.lax.rsqrt(v[...] + BN_EPS)
        bias = b[...] - m[...] * scale
        if i not in f32_bias:
            bias = bias.astype(jnp.bfloat16)
        folded.append(((w[...] * scale).astype(jnp.bfloat16), bias))
    oi = 0
    done = set()
    for i in range(n_convs):
        if i in done:
            continue
        j = cat_pairs.get(i)
        if j is None:
            outs[oi][...] = folded[i][0]
            outs[oi + 1][...] = folded[i][1]
        else:
            na = folded[i][0].shape[1]
            outs[oi][:, :na] = folded[i][0]
            outs[oi][:, na:] = folded[j][0]
            outs[oi + 1][:, :na] = folded[i][1]
            outs[oi + 1][:, na:] = folded[j][1]
            done.add(j)
        oi += 2


def _fold_all(groups, cat_pairs, f32_bias):
    n = len(groups)
    args = []
    out_shapes = []
    done = set()
    for i, (w, g, b, m, v) in enumerate(groups):
        r = (1, w.shape[1])
        bdt = jnp.float32 if i in f32_bias else jnp.bfloat16
        args += [w, g.reshape(r), b.reshape(r), m.reshape(r), v.reshape(r)]
        if i in done:
            continue
        j = cat_pairs.get(i)
        if j is None:
            out_shapes += [jax.ShapeDtypeStruct(w.shape, jnp.bfloat16),
                           jax.ShapeDtypeStruct(r, bdt)]
        else:
            wj = groups[j][0]
            nc = w.shape[1] + wj.shape[1]
            out_shapes += [jax.ShapeDtypeStruct((w.shape[0], nc), jnp.bfloat16),
                           jax.ShapeDtypeStruct((1, nc), bdt)]
            done.add(j)
    outs = pl.pallas_call(
        functools.partial(_fold_kernel, n_convs=n, cat_pairs=cat_pairs,
                          f32_bias=f32_bias),
        out_shape=tuple(out_shapes),
    )(*args)
    res = []
    for k in range(0, len(outs), 2):
        res.append((outs[k], outs[k + 1]))
    return res


def _im2col_conv3x3(act_bf16, col_ref, w2_ref, b2_ref, *, ipb, H, W, P):
    ab = act_bf16.reshape(ipb, H, W, P)
    for img in range(ipb):
        for ky in range(3):
            for kx in range(3):
                dy, dx = ky - 1, kx - 1
                h0, h1 = max(0, -dy), H - max(0, dy)
                w0, w1 = max(0, -dx), W - max(0, dx)
                t = ky * 3 + kx
                col_ref[img, h0:h1, w0:w1, t * P:(t + 1) * P] = (
                    ab[img, h0 + dy:h1 + dy, w0 + dx:w1 + dx, :])
    col = col_ref[...].reshape(ipb * H * W, 9 * P)
    acc = jnp.dot(col, w2_ref[...], preferred_element_type=jnp.float32)
    return jnp.maximum(acc.astype(jnp.bfloat16) + b2_ref[...], 0)


def _mega_kernel(x_ref,
                 ws_ref, bs_ref,
                 w0c_ref, b0c_ref, w02_ref, b02_ref, w03_ref, b03_ref,
                 w11_ref, b11_ref, w12_ref, b12_ref, w13_ref, b13_ref,
                 wc2_ref, bc2_ref, wc3_ref, bc3_ref,
                 w4_ref, b4_ref,
                 o_ref, col_ref, *, ipb, H, W, P):
    @pl.when(pl.program_id(0) == 0)
    def _init():
        zrow = jnp.zeros((1, W, P), jnp.bfloat16)
        zcol = jnp.zeros((H, 1, P), jnp.bfloat16)
        for img in range(ipb):
            for ky in range(3):
                for kx in range(3):
                    dy, dx = ky - 1, kx - 1
                    t = ky * 3 + kx
                    if dy == -1:
                        col_ref[img, 0:1, :, t * P:(t + 1) * P] = zrow
                    elif dy == 1:
                        col_ref[img, H - 1:H, :, t * P:(t + 1) * P] = zrow
                    if dx == -1:
                        col_ref[img, :, 0:1, t * P:(t + 1) * P] = zcol
                    elif dx == 1:
                        col_ref[img, :, W - 1:W, t * P:(t + 1) * P] = zcol

    xb = x_ref[...].astype(jnp.bfloat16)

    h1 = jnp.dot(xb, ws_ref[...], preferred_element_type=jnp.float32)
    h1 = _leaky(h1.astype(jnp.bfloat16) + bs_ref[...])

    t = jnp.dot(h1, w0c_ref[...], preferred_element_type=jnp.float32)
    t = t.astype(jnp.bfloat16) + b0c_ref[...]
    a0 = jnp.maximum(t[:, :P], 0)
    ident0 = t[:, P:]
    t2 = _im2col_conv3x3(a0, col_ref, w02_ref, b02_ref, ipb=ipb, H=H, W=W, P=P)
    t3 = jnp.dot(t2, w03_ref[...], preferred_element_type=jnp.float32)
    out0 = jnp.maximum(t3.astype(jnp.bfloat16) + b03_ref[...] + ident0, 0)

    a1 = jnp.dot(out0, w11_ref[...], preferred_element_type=jnp.float32)
    a1 = jnp.maximum(a1.astype(jnp.bfloat16) + b11_ref[...], 0)
    t2 = _im2col_conv3x3(a1, col_ref, w12_ref, b12_ref, ipb=ipb, H=H, W=W, P=P)
    t3 = jnp.dot(t2, w13_ref[...], preferred_element_type=jnp.float32)
    out1 = jnp.maximum(t3.astype(jnp.bfloat16) + b13_ref[...] + out0, 0)

    o2 = _leaky(jnp.dot(out1, wc2_ref[...],
                        preferred_element_type=jnp.float32).astype(jnp.bfloat16)
                + bc2_ref[...])
    o3 = _leaky(jnp.dot(xb, wc3_ref[...],
                        preferred_element_type=jnp.float32).astype(jnp.bfloat16)
                + bc3_ref[...])

    C2 = o_ref.shape[-1]
    y = jnp.dot(o2, w4_ref[:C2], preferred_element_type=jnp.float32)
    y = y + jnp.dot(o3, w4_ref[C2:], preferred_element_type=jnp.float32)
    o_ref[...] = _leaky(y + b4_ref[...])


def kernel(x, conv1_w, bn1_gamma, bn1_beta, bn1_mean, bn1_var, conv2_w, bn2_gamma, bn2_beta, bn2_mean, bn2_var, conv3_w, bn3_gamma, bn3_beta, bn3_mean, bn3_var, conv4_w, bn4_gamma, bn4_beta, bn4_mean, bn4_var, b0_conv1_w, b0_bn1_gamma, b0_bn1_beta, b0_bn1_mean, b0_bn1_var, b0_conv2_w, b0_bn2_gamma, b0_bn2_beta, b0_bn2_mean, b0_bn2_var, b0_conv3_w, b0_bn3_gamma, b0_bn3_beta, b0_bn3_mean, b0_bn3_var, b0_down_w, b0_down_bn_gamma, b0_down_bn_beta, b0_down_bn_mean, b0_down_bn_var, b1_conv1_w, b1_bn1_gamma, b1_bn1_beta, b1_bn1_mean, b1_bn1_var, b1_conv2_w, b1_bn2_gamma, b1_bn2_beta, b1_bn2_mean, b1_bn2_var, b1_conv3_w, b1_bn3_gamma, b1_bn3_beta, b1_bn3_mean, b1_bn3_var):
    N, H, W, P = x.shape
    HW = H * W
    C2 = 2 * P

    groups = [
        (conv1_w, bn1_gamma, bn1_beta, bn1_mean, bn1_var),
        (b0_conv1_w, b0_bn1_gamma, b0_bn1_beta, b0_bn1_mean, b0_bn1_var),
        (b0_down_w, b0_down_bn_gamma, b0_down_bn_beta, b0_down_bn_mean,
         b0_down_bn_var),
        (b0_conv2_w.reshape(9 * P, P), b0_bn2_gamma, b0_bn2_beta,
         b0_bn2_mean, b0_bn2_var),
        (b0_conv3_w, b0_bn3_gamma, b0_bn3_beta, b0_bn3_mean, b0_bn3_var),
        (b1_conv1_w, b1_bn1_gamma, b1_bn1_beta, b1_bn1_mean, b1_bn1_var),
        (b1_conv2_w.reshape(9 * P, P), b1_bn2_gamma, b1_bn2_beta,
         b1_bn2_mean, b1_bn2_var),
        (b1_conv3_w, b1_bn3_gamma, b1_bn3_beta, b1_bn3_mean, b1_bn3_var),
        (conv2_w, bn2_gamma, bn2_beta, bn2_mean, bn2_var),
        (conv3_w, bn3_gamma, bn3_beta, bn3_mean, bn3_var),
        (conv4_w, bn4_gamma, bn4_beta, bn4_mean, bn4_var),
    ]
    folded = _fold_all(groups, cat_pairs={1: 2}, f32_bias={10})
    (ws, bs), (w0c, b0c), (w02, b02), (w03, b03), (w11, b11), \
        (w12, b12), (w13, b13), (wc2, bc2), (wc3, bc3), (w4, b4) = folded

    ipb = IMAGES_PER_STEP if N % IMAGES_PER_STEP == 0 else 1
    rows = ipb * HW
    x2d = x.reshape(N * HW, P)

    def c(shape):
        nd = len(shape)
        return pl.BlockSpec(shape, lambda i, nd=nd: (0,) * nd)

    y2d = pl.pallas_call(
        functools.partial(_mega_kernel, ipb=ipb, H=H, W=W, P=P),
        grid=(N // ipb,),
        in_specs=[
            pl.BlockSpec((rows, P), lambda i: (i, 0)),
            c(ws.shape), c(bs.shape),
            c(w0c.shape), c(b0c.shape), c(w02.shape), c(b02.shape),
            c(w03.shape), c(b03.shape),
            c(w11.shape), c(b11.shape), c(w12.shape), c(b12.shape),
            c(w13.shape), c(b13.shape),
            c(wc2.shape), c(bc2.shape), c(wc3.shape), c(bc3.shape),
            c(w4.shape), c(b4.shape),
        ],
        out_specs=pl.BlockSpec((rows, C2), lambda i: (i, 0)),
        out_shape=jax.ShapeDtypeStruct((N * HW, C2), jnp.float32),
        scratch_shapes=[pltpu.VMEM((ipb, H, W, 9 * P), jnp.bfloat16)],
        compiler_params=pltpu.CompilerParams(dimension_semantics=("arbitrary",)),
    )(x2d, ws, bs, w0c, b0c, w02, b02, w03, b03,
      w11, b11, w12, b12, w13, b13, wc2, bc2, wc3, bc3, w4, b4)
    return y2d.reshape(N, H, W, C2)


# speedup vs baseline: 1.0613x; 1.0613x over previous
import functools

import jax
import jax.numpy as jnp
from jax.experimental import pallas as pl
from jax.experimental.pallas import tpu as pltpu

LEAKY_SLOPE = 0.1
BN_EPS = 1e-5
IMAGES_PER_STEP = 4


def _leaky(v):
    return jnp.maximum(v, LEAKY_SLOPE * v)


def _fold_kernel(*refs, n_convs, cat_pairs, f32_bias):
    ins = refs[:5 * n_convs]
    outs = refs[5 * n_convs:]
    folded = []
    for i in range(n_convs):
        w, g, b, m, v = ins[5 * i:5 * i + 5]
        scale = g[...] * jax.lax.rsqrt(v[...] + BN_EPS)
        bias = b[...] - m[...] * scale
        if i not in f32_bias:
            bias = bias.astype(jnp.bfloat16)
        folded.append(((w[...] * scale).astype(jnp.bfloat16), bias))
    oi = 0
    done = set()
    for i in range(n_convs):
        if i in done:
            continue
        j = cat_pairs.get(i)
        if j is None:
            outs[oi][...] = folded[i][0]
            outs[oi + 1][...] = folded[i][1]
        else:
            na = folded[i][0].shape[1]
            outs[oi][:, :na] = folded[i][0]
            outs[oi][:, na:] = folded[j][0]
            outs[oi + 1][:, :na] = folded[i][1]
            outs[oi + 1][:, na:] = folded[j][1]
            done.add(j)
        oi += 2


def _fold_all(groups, cat_pairs, f32_bias):
    n = len(groups)
    args = []
    out_shapes = []
    done = set()
    for i, (w, g, b, m, v) in enumerate(groups):
        r = (1, w.shape[1])
        bdt = jnp.float32 if i in f32_bias else jnp.bfloat16
        args += [w, g.reshape(r), b.reshape(r), m.reshape(r), v.reshape(r)]
        if i in done:
            continue
        j = cat_pairs.get(i)
        if j is None:
            out_shapes += [jax.ShapeDtypeStruct(w.shape, jnp.bfloat16),
                           jax.ShapeDtypeStruct(r, bdt)]
        else:
            wj = groups[j][0]
            nc = w.shape[1] + wj.shape[1]
            out_shapes += [jax.ShapeDtypeStruct((w.shape[0], nc), jnp.bfloat16),
                           jax.ShapeDtypeStruct((1, nc), bdt)]
            done.add(j)
    outs = pl.pallas_call(
        functools.partial(_fold_kernel, n_convs=n, cat_pairs=cat_pairs,
                          f32_bias=f32_bias),
        out_shape=tuple(out_shapes),
    )(*args)
    res = []
    for k in range(0, len(outs), 2):
        res.append((outs[k], outs[k + 1]))
    return res


def _im2col_conv3x3(act_bf16, col_ref, w2_ref, b2_ref, *, ipb, H, W, P):
    ab = act_bf16.reshape(ipb, H, W, P)
    for img in range(ipb):
        for ky in range(3):
            for kx in range(3):
                dy, dx = ky - 1, kx - 1
                h0, h1 = max(0, -dy), H - max(0, dy)
                w0, w1 = max(0, -dx), W - max(0, dx)
                t = ky * 3 + kx
                col_ref[img, h0:h1, w0:w1, t * P:(t + 1) * P] = (
                    ab[img, h0 + dy:h1 + dy, w0 + dx:w1 + dx, :])
    col = col_ref[...].reshape(ipb * H * W, 9 * P)
    acc = jnp.dot(col, w2_ref[...], preferred_element_type=jnp.float32)
    return jnp.maximum(acc.astype(jnp.bfloat16) + b2_ref[...], 0)


def _border_init(col_ref, *, ipb, H, W, P):
    zrow = jnp.zeros((1, W, P), jnp.bfloat16)
    zcol = jnp.zeros((H, 1, P), jnp.bfloat16)
    for img in range(ipb):
        for ky in range(3):
            for kx in range(3):
                dy, dx = ky - 1, kx - 1
                t = ky * 3 + kx
                if dy == -1:
                    col_ref[img, 0:1, :, t * P:(t + 1) * P] = zrow
                elif dy == 1:
                    col_ref[img, H - 1:H, :, t * P:(t + 1) * P] = zrow
                if dx == -1:
                    col_ref[img, :, 0:1, t * P:(t + 1) * P] = zcol
                elif dx == 1:
                    col_ref[img, :, W - 1:W, t * P:(t + 1) * P] = zcol


def _chain(xb, ws_ref, bs_ref,
           w0c_ref, b0c_ref, w02_ref, b02_ref, w03_ref, b03_ref,
           w11_ref, b11_ref, w12_ref, b12_ref, w13_ref, b13_ref,
           wc2_ref, bc2_ref, wc3_ref, bc3_ref, w4_ref, b4_ref,
           col_ref, *, ipb, H, W, P):
    h1 = jnp.dot(xb, ws_ref[...], preferred_element_type=jnp.float32)
    h1 = _leaky(h1.astype(jnp.bfloat16) + bs_ref[...])

    t = jnp.dot(h1, w0c_ref[...], preferred_element_type=jnp.float32)
    t = t.astype(jnp.bfloat16) + b0c_ref[...]
    a0 = jnp.maximum(t[:, :P], 0)
    ident0 = t[:, P:]
    t2 = _im2col_conv3x3(a0, col_ref, w02_ref, b02_ref, ipb=ipb, H=H, W=W, P=P)
    t3 = jnp.dot(t2, w03_ref[...], preferred_element_type=jnp.float32)
    out0 = jnp.maximum(t3.astype(jnp.bfloat16) + b03_ref[...] + ident0, 0)

    a1 = jnp.dot(out0, w11_ref[...], preferred_element_type=jnp.float32)
    a1 = jnp.maximum(a1.astype(jnp.bfloat16) + b11_ref[...], 0)
    t2 = _im2col_conv3x3(a1, col_ref, w12_ref, b12_ref, ipb=ipb, H=H, W=W, P=P)
    t3 = jnp.dot(t2, w13_ref[...], preferred_element_type=jnp.float32)
    out1 = jnp.maximum(t3.astype(jnp.bfloat16) + b13_ref[...] + out0, 0)

    o2 = _leaky(jnp.dot(out1, wc2_ref[...],
                        preferred_element_type=jnp.float32).astype(jnp.bfloat16)
                + bc2_ref[...])
    o3 = _leaky(jnp.dot(xb, wc3_ref[...],
                        preferred_element_type=jnp.float32).astype(jnp.bfloat16)
                + bc3_ref[...])

    C2 = 2 * P
    y = jnp.dot(o2, w4_ref[:C2], preferred_element_type=jnp.float32)
    y = y + jnp.dot(o3, w4_ref[C2:], preferred_element_type=jnp.float32)
    return _leaky(y + b4_ref[...])


def _mega_kernel(x_ref,
                 ws_ref, bs_ref,
                 w0c_ref, b0c_ref, w02_ref, b02_ref, w03_ref, b03_ref,
                 w11_ref, b11_ref, w12_ref, b12_ref, w13_ref, b13_ref,
                 wc2_ref, bc2_ref, wc3_ref, bc3_ref,
                 w4_ref, b4_ref,
                 o_ref, *col_refs, ipb, H, W, P):
    ipbs = [ipb // 2, ipb - ipb // 2] if ipb >= 2 else [ipb]

    @pl.when(pl.program_id(0) == 0)
    def _init():
        for cref, n in zip(col_refs, ipbs):
            _border_init(cref, ipb=n, H=H, W=W, P=P)

    wargs = (ws_ref, bs_ref, w0c_ref, b0c_ref, w02_ref, b02_ref, w03_ref,
             b03_ref, w11_ref, b11_ref, w12_ref, b12_ref, w13_ref, b13_ref,
             wc2_ref, bc2_ref, wc3_ref, bc3_ref, w4_ref, b4_ref)
    xb = x_ref[...].astype(jnp.bfloat16)
    row = 0
    for cref, n in zip(col_refs, ipbs):
        nrows = n * H * W
        y = _chain(xb[row:row + nrows], *wargs, cref, ipb=n, H=H, W=W, P=P)
        o_ref[row:row + nrows] = y
        row += nrows


def kernel(x, conv1_w, bn1_gamma, bn1_beta, bn1_mean, bn1_var, conv2_w, bn2_gamma, bn2_beta, bn2_mean, bn2_var, conv3_w, bn3_gamma, bn3_beta, bn3_mean, bn3_var, conv4_w, bn4_gamma, bn4_beta, bn4_mean, bn4_var, b0_conv1_w, b0_bn1_gamma, b0_bn1_beta, b0_bn1_mean, b0_bn1_var, b0_conv2_w, b0_bn2_gamma, b0_bn2_beta, b0_bn2_mean, b0_bn2_var, b0_conv3_w, b0_bn3_gamma, b0_bn3_beta, b0_bn3_mean, b0_bn3_var, b0_down_w, b0_down_bn_gamma, b0_down_bn_beta, b0_down_bn_mean, b0_down_bn_var, b1_conv1_w, b1_bn1_gamma, b1_bn1_beta, b1_bn1_mean, b1_bn1_var, b1_conv2_w, b1_bn2_gamma, b1_bn2_beta, b1_bn2_mean, b1_bn2_var, b1_conv3_w, b1_bn3_gamma, b1_bn3_beta, b1_bn3_mean, b1_bn3_var):
    N, H, W, P = x.shape
    HW = H * W
    C2 = 2 * P

    groups = [
        (conv1_w, bn1_gamma, bn1_beta, bn1_mean, bn1_var),
        (b0_conv1_w, b0_bn1_gamma, b0_bn1_beta, b0_bn1_mean, b0_bn1_var),
        (b0_down_w, b0_down_bn_gamma, b0_down_bn_beta, b0_down_bn_mean,
         b0_down_bn_var),
        (b0_conv2_w.reshape(9 * P, P), b0_bn2_gamma, b0_bn2_beta,
         b0_bn2_mean, b0_bn2_var),
        (b0_conv3_w, b0_bn3_gamma, b0_bn3_beta, b0_bn3_mean, b0_bn3_var),
        (b1_conv1_w, b1_bn1_gamma, b1_bn1_beta, b1_bn1_mean, b1_bn1_var),
        (b1_conv2_w.reshape(9 * P, P), b1_bn2_gamma, b1_bn2_beta,
         b1_bn2_mean, b1_bn2_var),
        (b1_conv3_w, b1_bn3_gamma, b1_bn3_beta, b1_bn3_mean, b1_bn3_var),
        (conv2_w, bn2_gamma, bn2_beta, bn2_mean, bn2_var),
        (conv3_w, bn3_gamma, bn3_beta, bn3_mean, bn3_var),
        (conv4_w, bn4_gamma, bn4_beta, bn4_mean, bn4_var),
    ]
    folded = _fold_all(groups, cat_pairs={1: 2}, f32_bias={10})
    (ws, bs), (w0c, b0c), (w02, b02), (w03, b03), (w11, b11), \
        (w12, b12), (w13, b13), (wc2, bc2), (wc3, bc3), (w4, b4) = folded

    ipb = next(n for n in (IMAGES_PER_STEP, 2, 1) if N % n == 0)
    rows = ipb * HW
    x2d = x.reshape(N * HW, P)

    def c(shape):
        nd = len(shape)
        return pl.BlockSpec(shape, lambda i, nd=nd: (0,) * nd)

    y2d = pl.pallas_call(
        functools.partial(_mega_kernel, ipb=ipb, H=H, W=W, P=P),
        grid=(N // ipb,),
        in_specs=[
            pl.BlockSpec((rows, P), lambda i: (i, 0)),
            c(ws.shape), c(bs.shape),
            c(w0c.shape), c(b0c.shape), c(w02.shape), c(b02.shape),
            c(w03.shape), c(b03.shape),
            c(w11.shape), c(b11.shape), c(w12.shape), c(b12.shape),
            c(w13.shape), c(b13.shape),
            c(wc2.shape), c(bc2.shape), c(wc3.shape), c(bc3.shape),
            c(w4.shape), c(b4.shape),
        ],
        out_specs=pl.BlockSpec((rows, C2), lambda i: (i, 0)),
        out_shape=jax.ShapeDtypeStruct((N * HW, C2), jnp.float32),
        scratch_shapes=(
            [pltpu.VMEM((ipb // 2, H, W, 9 * P), jnp.bfloat16),
             pltpu.VMEM((ipb - ipb // 2, H, W, 9 * P), jnp.bfloat16)]
            if ipb >= 2 else [pltpu.VMEM((ipb, H, W, 9 * P), jnp.bfloat16)]),
        compiler_params=pltpu.CompilerParams(dimension_semantics=("arbitrary",)),
    )(x2d, ws, bs, w0c, b0c, w02, b02, w03, b03,
      w11, b11, w12, b12, w13, b13, wc2, bc2, wc3, bc3, w4, b4)
    return y2d.reshape(N, H, W, C2)


# speedup vs baseline: 1.0787x; 1.0164x over previous
import functools

import jax
import jax.numpy as jnp
from jax.experimental import pallas as pl
from jax.experimental.pallas import tpu as pltpu

LEAKY_SLOPE = 0.1
BN_EPS = 1e-5
IMAGES_PER_STEP = 8


def _leaky(v):
    return jnp.maximum(v, LEAKY_SLOPE * v)


def _fold_kernel(*refs, n_convs, cat_pairs, f32_bias):
    ins = refs[:5 * n_convs]
    outs = refs[5 * n_convs:]
    folded = []
    for i in range(n_convs):
        w, g, b, m, v = ins[5 * i:5 * i + 5]
        scale = g[...] * jax.lax.rsqrt(v[...] + BN_EPS)
        bias = b[...] - m[...] * scale
        if i not in f32_bias:
            bias = bias.astype(jnp.bfloat16)
        folded.append(((w[...] * scale).astype(jnp.bfloat16), bias))
    oi = 0
    done = set()
    for i in range(n_convs):
        if i in done:
            continue
        j = cat_pairs.get(i)
        if j is None:
            outs[oi][...] = folded[i][0]
            outs[oi + 1][...] = folded[i][1]
        else:
            na = folded[i][0].shape[1]
            outs[oi][:, :na] = folded[i][0]
            outs[oi][:, na:] = folded[j][0]
            outs[oi + 1][:, :na] = folded[i][1]
            outs[oi + 1][:, na:] = folded[j][1]
            done.add(j)
        oi += 2


def _fold_all(groups, cat_pairs, f32_bias):
    n = len(groups)
    args = []
    out_shapes = []
    done = set()
    for i, (w, g, b, m, v) in enumerate(groups):
        r = (1, w.shape[1])
        bdt = jnp.float32 if i in f32_bias else jnp.bfloat16
        args += [w, g.reshape(r), b.reshape(r), m.reshape(r), v.reshape(r)]
        if i in done:
            continue
        j = cat_pairs.get(i)
        if j is None:
            out_shapes += [jax.ShapeDtypeStruct(w.shape, jnp.bfloat16),
                           jax.ShapeDtypeStruct(r, bdt)]
        else:
            wj = groups[j][0]
            nc = w.shape[1] + wj.shape[1]
            out_shapes += [jax.ShapeDtypeStruct((w.shape[0], nc), jnp.bfloat16),
                           jax.ShapeDtypeStruct((1, nc), bdt)]
            done.add(j)
    outs = pl.pallas_call(
        functools.partial(_fold_kernel, n_convs=n, cat_pairs=cat_pairs,
                          f32_bias=f32_bias),
        out_shape=tuple(out_shapes),
    )(*args)
    res = []
    for k in range(0, len(outs), 2):
        res.append((outs[k], outs[k + 1]))
    return res


def _im2col_conv3x3(act_bf16, col_ref, w2_ref, b2_ref, *, ipb, H, W, P):
    ab = act_bf16.reshape(ipb, H, W, P)
    for img in range(ipb):
        for ky in range(3):
            for kx in range(3):
                dy, dx = ky - 1, kx - 1
                h0, h1 = max(0, -dy), H - max(0, dy)
                w0, w1 = max(0, -dx), W - max(0, dx)
                t = ky * 3 + kx
                col_ref[img, h0:h1, w0:w1, t * P:(t + 1) * P] = (
                    ab[img, h0 + dy:h1 + dy, w0 + dx:w1 + dx, :])
    col = col_ref[...].reshape(ipb * H * W, 9 * P)
    acc = jnp.dot(col, w2_ref[...], preferred_element_type=jnp.float32)
    return jnp.maximum(acc.astype(jnp.bfloat16) + b2_ref[...], 0)


def _chain_split(ipb):
    if ipb >= 2:
        return [ipb // 2, ipb - ipb // 2]
    return [ipb]


def _border_init(col_ref, *, ipb, H, W, P):
    zrow = jnp.zeros((1, W, P), jnp.bfloat16)
    zcol = jnp.zeros((H, 1, P), jnp.bfloat16)
    for img in range(ipb):
        for ky in range(3):
            for kx in range(3):
                dy, dx = ky - 1, kx - 1
                t = ky * 3 + kx
                if dy == -1:
                    col_ref[img, 0:1, :, t * P:(t + 1) * P] = zrow
                elif dy == 1:
                    col_ref[img, H - 1:H, :, t * P:(t + 1) * P] = zrow
                if dx == -1:
                    col_ref[img, :, 0:1, t * P:(t + 1) * P] = zcol
                elif dx == 1:
                    col_ref[img, :, W - 1:W, t * P:(t + 1) * P] = zcol


def _chain(xb, ws_ref, bs_ref,
           w0c_ref, b0c_ref, w02_ref, b02_ref, w03_ref, b03_ref,
           w11_ref, b11_ref, w12_ref, b12_ref, w13_ref, b13_ref,
           wc2_ref, bc2_ref, wc3_ref, bc3_ref, w4_ref, b4_ref,
           col_ref, *, ipb, H, W, P):
    h1 = jnp.dot(xb, ws_ref[...], preferred_element_type=jnp.float32)
    h1 = _leaky(h1.astype(jnp.bfloat16) + bs_ref[...])

    t = jnp.dot(h1, w0c_ref[...], preferred_element_type=jnp.float32)
    t = t.astype(jnp.bfloat16) + b0c_ref[...]
    a0 = jnp.maximum(t[:, :P], 0)
    ident0 = t[:, P:]
    t2 = _im2col_conv3x3(a0, col_ref, w02_ref, b02_ref, ipb=ipb, H=H, W=W, P=P)
    t3 = jnp.dot(t2, w03_ref[...], preferred_element_type=jnp.float32)
    out0 = jnp.maximum(t3.astype(jnp.bfloat16) + b03_ref[...] + ident0, 0)

    a1 = jnp.dot(out0, w11_ref[...], preferred_element_type=jnp.float32)
    a1 = jnp.maximum(a1.astype(jnp.bfloat16) + b11_ref[...], 0)
    t2 = _im2col_conv3x3(a1, col_ref, w12_ref, b12_ref, ipb=ipb, H=H, W=W, P=P)
    t3 = jnp.dot(t2, w13_ref[...], preferred_element_type=jnp.float32)
    out1 = jnp.maximum(t3.astype(jnp.bfloat16) + b13_ref[...] + out0, 0)

    o2 = _leaky(jnp.dot(out1, wc2_ref[...],
                        preferred_element_type=jnp.float32).astype(jnp.bfloat16)
                + bc2_ref[...])
    o3 = _leaky(jnp.dot(xb, wc3_ref[...],
                        preferred_element_type=jnp.float32).astype(jnp.bfloat16)
                + bc3_ref[...])

    C2 = 2 * P
    y = jnp.dot(o2, w4_ref[:C2], preferred_element_type=jnp.float32)
    y = y + jnp.dot(o3, w4_ref[C2:], preferred_element_type=jnp.float32)
    return _leaky(y + b4_ref[...])


def _mega_kernel(x_ref,
                 ws_ref, bs_ref,
                 w0c_ref, b0c_ref, w02_ref, b02_ref, w03_ref, b03_ref,
                 w11_ref, b11_ref, w12_ref, b12_ref, w13_ref, b13_ref,
                 wc2_ref, bc2_ref, wc3_ref, bc3_ref,
                 w4_ref, b4_ref,
                 o_ref, *col_refs, ipb, H, W, P):
    ipbs = _chain_split(ipb)

    @pl.when(pl.program_id(0) == 0)
    def _init():
        for cref, n in zip(col_refs, ipbs):
            _border_init(cref, ipb=n, H=H, W=W, P=P)

    wargs = (ws_ref, bs_ref, w0c_ref, b0c_ref, w02_ref, b02_ref, w03_ref,
             b03_ref, w11_ref, b11_ref, w12_ref, b12_ref, w13_ref, b13_ref,
             wc2_ref, bc2_ref, wc3_ref, bc3_ref, w4_ref, b4_ref)
    xb = x_ref[...].astype(jnp.bfloat16)
    row = 0
    for cref, n in zip(col_refs, ipbs):
        nrows = n * H * W
        y = _chain(xb[row:row + nrows], *wargs, cref, ipb=n, H=H, W=W, P=P)
        o_ref[row:row + nrows] = y
        row += nrows


def kernel(x, conv1_w, bn1_gamma, bn1_beta, bn1_mean, bn1_var, conv2_w, bn2_gamma, bn2_beta, bn2_mean, bn2_var, conv3_w, bn3_gamma, bn3_beta, bn3_mean, bn3_var, conv4_w, bn4_gamma, bn4_beta, bn4_mean, bn4_var, b0_conv1_w, b0_bn1_gamma, b0_bn1_beta, b0_bn1_mean, b0_bn1_var, b0_conv2_w, b0_bn2_gamma, b0_bn2_beta, b0_bn2_mean, b0_bn2_var, b0_conv3_w, b0_bn3_gamma, b0_bn3_beta, b0_bn3_mean, b0_bn3_var, b0_down_w, b0_down_bn_gamma, b0_down_bn_beta, b0_down_bn_mean, b0_down_bn_var, b1_conv1_w, b1_bn1_gamma, b1_bn1_beta, b1_bn1_mean, b1_bn1_var, b1_conv2_w, b1_bn2_gamma, b1_bn2_beta, b1_bn2_mean, b1_bn2_var, b1_conv3_w, b1_bn3_gamma, b1_bn3_beta, b1_bn3_mean, b1_bn3_var):
    N, H, W, P = x.shape
    HW = H * W
    C2 = 2 * P

    groups = [
        (conv1_w, bn1_gamma, bn1_beta, bn1_mean, bn1_var),
        (b0_conv1_w, b0_bn1_gamma, b0_bn1_beta, b0_bn1_mean, b0_bn1_var),
        (b0_down_w, b0_down_bn_gamma, b0_down_bn_beta, b0_down_bn_mean,
         b0_down_bn_var),
        (b0_conv2_w.reshape(9 * P, P), b0_bn2_gamma, b0_bn2_beta,
         b0_bn2_mean, b0_bn2_var),
        (b0_conv3_w, b0_bn3_gamma, b0_bn3_beta, b0_bn3_mean, b0_bn3_var),
        (b1_conv1_w, b1_bn1_gamma, b1_bn1_beta, b1_bn1_mean, b1_bn1_var),
        (b1_conv2_w.reshape(9 * P, P), b1_bn2_gamma, b1_bn2_beta,
         b1_bn2_mean, b1_bn2_var),
        (b1_conv3_w, b1_bn3_gamma, b1_bn3_beta, b1_bn3_mean, b1_bn3_var),
        (conv2_w, bn2_gamma, bn2_beta, bn2_mean, bn2_var),
        (conv3_w, bn3_gamma, bn3_beta, bn3_mean, bn3_var),
        (conv4_w, bn4_gamma, bn4_beta, bn4_mean, bn4_var),
    ]
    folded = _fold_all(groups, cat_pairs={1: 2}, f32_bias={10})
    (ws, bs), (w0c, b0c), (w02, b02), (w03, b03), (w11, b11), \
        (w12, b12), (w13, b13), (wc2, bc2), (wc3, bc3), (w4, b4) = folded

    ipb = next(n for n in (IMAGES_PER_STEP, 4, 2, 1) if N % n == 0)
    rows = ipb * HW
    x2d = x.reshape(N * HW, P)

    def c(shape):
        nd = len(shape)
        return pl.BlockSpec(shape, lambda i, nd=nd: (0,) * nd)

    y2d = pl.pallas_call(
        functools.partial(_mega_kernel, ipb=ipb, H=H, W=W, P=P),
        grid=(N // ipb,),
        in_specs=[
            pl.BlockSpec((rows, P), lambda i: (i, 0)),
            c(ws.shape), c(bs.shape),
            c(w0c.shape), c(b0c.shape), c(w02.shape), c(b02.shape),
            c(w03.shape), c(b03.shape),
            c(w11.shape), c(b11.shape), c(w12.shape), c(b12.shape),
            c(w13.shape), c(b13.shape),
            c(wc2.shape), c(bc2.shape), c(wc3.shape), c(bc3.shape),
            c(w4.shape), c(b4.shape),
        ],
        out_specs=pl.BlockSpec((rows, C2), lambda i: (i, 0)),
        out_shape=jax.ShapeDtypeStruct((N * HW, C2), jnp.float32),
        scratch_shapes=[pltpu.VMEM((n, H, W, 9 * P), jnp.bfloat16)
                        for n in _chain_split(ipb)],
        compiler_params=pltpu.CompilerParams(dimension_semantics=("arbitrary",)),
    )(x2d, ws, bs, w0c, b0c, w02, b02, w03, b03,
      w11, b11, w12, b12, w13, b13, wc2, bc2, wc3, bc3, w4, b4)
    return y2d.reshape(N, H, W, C2)
